# baseline (device time: 377566 ns/iter reference)
import jax
import jax.numpy as jnp
from jax import lax
from jax.experimental import pallas as pl
from jax.experimental.pallas import tpu as pltpu

N_DEV = 4
N_PER = 2048
D = 1024
E = 32
E_LOC = 8
CAP = 204
CAPP = 208
SLOTS = E_LOC * CAPP
R_ROWS = N_PER // 128


def _ring_ag(x, r2):

    def body(x_ref, r_ref, xall_ref, rall_ref, sx, rx, sr, rr):
        my = lax.axis_index("i")
        left = lax.rem(my + N_DEV - 1, N_DEV)
        right = lax.rem(my + 1, N_DEV)

        barrier_sem = pltpu.get_barrier_semaphore()
        for nbr in (left, right):
            pl.semaphore_signal(
                barrier_sem, inc=1, device_id=(nbr,),
                device_id_type=pl.DeviceIdType.MESH,
            )
        pl.semaphore_wait(barrier_sem, 2)

        xall_ref[pl.ds(my * N_PER, N_PER), :] = x_ref[:, :].astype(jnp.bfloat16)
        rall_ref[pl.ds(my * R_ROWS, R_ROWS), :] = r_ref[:, :]

        for h in range(N_DEV - 1):
            sb = lax.rem(my + N_DEV - h, N_DEV)
            rdma_x = pltpu.make_async_remote_copy(
                src_ref=xall_ref.at[pl.ds(sb * N_PER, N_PER), :],
                dst_ref=xall_ref.at[pl.ds(sb * N_PER, N_PER), :],
                send_sem=sx.at[h],
                recv_sem=rx.at[h],
                device_id=(right,),
                device_id_type=pl.DeviceIdType.MESH,
            )
            rdma_r = pltpu.make_async_remote_copy(
                src_ref=rall_ref.at[pl.ds(sb * R_ROWS, R_ROWS), :],
                dst_ref=rall_ref.at[pl.ds(sb * R_ROWS, R_ROWS), :],
                send_sem=sr.at[h],
                recv_sem=rr.at[h],
                device_id=(right,),
                device_id_type=pl.DeviceIdType.MESH,
            )
            rdma_x.start()
            rdma_r.start()
            rdma_x.wait()
            rdma_r.wait()

    return pl.pallas_call(
        body,
        out_shape=(
            jax.ShapeDtypeStruct((N_DEV * N_PER, D), jnp.bfloat16),
            jax.ShapeDtypeStruct((N_DEV * R_ROWS, 128), jnp.int32),
        ),
        in_specs=[
            pl.BlockSpec(memory_space=pltpu.VMEM),
            pl.BlockSpec(memory_space=pltpu.VMEM),
        ],
        out_specs=(
            pl.BlockSpec(memory_space=pltpu.VMEM),
            pl.BlockSpec(memory_space=pltpu.VMEM),
        ),
        scratch_shapes=[
            pltpu.SemaphoreType.DMA((N_DEV - 1,)),
            pltpu.SemaphoreType.DMA((N_DEV - 1,)),
            pltpu.SemaphoreType.DMA((N_DEV - 1,)),
            pltpu.SemaphoreType.DMA((N_DEV - 1,)),
        ],
        compiler_params=pltpu.CompilerParams(collective_id=0),
    )(x, r2)


def _expert_mm_ag(x_all, rr2, rk2, expert_W):

    def body(x_ref, rr_ref, rk_ref, ew_ref, yall_ref, wbuf, csem, ss, rs):
        my = lax.axis_index("i")
        left = lax.rem(my + N_DEV - 1, N_DEV)
        right = lax.rem(my + 1, N_DEV)

        barrier_sem = pltpu.get_barrier_semaphore()
        for nbr in (left, right):
            pl.semaphore_signal(
                barrier_sem, inc=1, device_id=(nbr,),
                device_id_type=pl.DeviceIdType.MESH,
            )
        pl.semaphore_wait(barrier_sem, 2)

        def expert_step(le, _):
            cp = pltpu.make_async_copy(ew_ref.at[le], wbuf, csem)
            cp.start()
            e = my * E_LOC + le
            rr = rr_ref[0:1, :]
            rk = rk_ref[0:1, :]
            c_iota = lax.broadcasted_iota(jnp.int32, (CAPP, N_DEV * N_PER), 0)
            mine = (rr == e) & (rk < CAP)
            onehot = ((c_iota == rk) & mine).astype(jnp.bfloat16)
            xd = jnp.dot(
                onehot, x_ref[:, :], preferred_element_type=jnp.float32
            ).astype(jnp.bfloat16)
            cp.wait()
            w16 = wbuf[:, :].astype(jnp.bfloat16)
            y = jnp.dot(xd, w16, preferred_element_type=jnp.float32)
            yall_ref[pl.ds(my * SLOTS + le * CAPP, CAPP), :] = y.astype(jnp.bfloat16)
            return 0

        lax.fori_loop(0, E_LOC, expert_step, 0)

        for h in range(N_DEV - 1):
            sb = lax.rem(my + N_DEV - h, N_DEV)
            rdma = pltpu.make_async_remote_copy(
                src_ref=yall_ref.at[pl.ds(sb * SLOTS, SLOTS), :],
                dst_ref=yall_ref.at[pl.ds(sb * SLOTS, SLOTS), :],
                send_sem=ss.at[h],
                recv_sem=rs.at[h],
                device_id=(right,),
                device_id_type=pl.DeviceIdType.MESH,
            )
            rdma.start()
            rdma.wait()

    return pl.pallas_call(
        body,
        out_shape=jax.ShapeDtypeStruct((N_DEV * SLOTS, D), jnp.bfloat16),
        in_specs=[
            pl.BlockSpec(memory_space=pltpu.VMEM),
            pl.BlockSpec(memory_space=pltpu.VMEM),
            pl.BlockSpec(memory_space=pltpu.VMEM),
            pl.BlockSpec(memory_space=pl.ANY),
        ],
        out_specs=pl.BlockSpec(memory_space=pltpu.VMEM),
        scratch_shapes=[
            pltpu.VMEM((D, D), jnp.float32),
            pltpu.SemaphoreType.DMA,
            pltpu.SemaphoreType.DMA((N_DEV - 1,)),
            pltpu.SemaphoreType.DMA((N_DEV - 1,)),
        ],
        compiler_params=pltpu.CompilerParams(collective_id=1),
    )(x_all, rr2, rk2, expert_W)


def _combine(y_all, gm2):

    def body(y_ref, gm_ref, out_ref):
        gmv = gm_ref[:, 0:1]
        out_ref[:, :] = jnp.zeros((N_PER, D), jnp.float32)

        def step(s, _):
            k = lax.broadcasted_iota(jnp.int32, (N_PER, SLOTS), 1) + s * SLOTS
            G = (gmv == k).astype(jnp.bfloat16)
            out_ref[:, :] += jnp.dot(
                G, y_ref[pl.ds(s * SLOTS, SLOTS), :],
                preferred_element_type=jnp.float32,
            )
            return 0

        lax.fori_loop(0, N_DEV, step, 0)

    return pl.pallas_call(
        body,
        out_shape=jax.ShapeDtypeStruct((N_PER, D), jnp.float32),
        in_specs=[
            pl.BlockSpec(memory_space=pltpu.VMEM),
            pl.BlockSpec(memory_space=pltpu.VMEM),
        ],
        out_specs=pl.BlockSpec(memory_space=pltpu.VMEM),
    )(y_all, gm2)


def kernel(x, router_W, route_idx, expert_W):
    del router_W
    my = lax.axis_index("i")

    x_all, rall = _ring_ag(x, route_idx.reshape(R_ROWS, 128))
    r_all = rall.reshape(N_DEV * N_PER)

    oh = (r_all[:, None] == jnp.arange(E, dtype=jnp.int32)[None, :]).astype(jnp.int32)
    rank = jnp.sum((jnp.cumsum(oh, axis=0) - 1) * oh, axis=1)

    rr2 = r_all.reshape(1, N_DEV * N_PER)
    rk2 = rank.reshape(1, N_DEV * N_PER)
    y_all = _expert_mm_ag(x_all, rr2, rk2, expert_W)

    r_mine = lax.dynamic_slice(r_all, (my * N_PER,), (N_PER,))
    rank_mine = lax.dynamic_slice(rank, (my * N_PER,), (N_PER,))
    kept_mine = rank_mine < CAP
    gm = jnp.where(kept_mine, r_mine * CAPP + rank_mine, N_DEV * SLOTS)
    return _combine(y_all, gm.reshape(N_PER, 1))


# device time: 250868 ns/iter; 1.5050x vs baseline; 1.5050x over previous
import jax
import jax.numpy as jnp
from jax import lax
from jax.experimental import pallas as pl
from jax.experimental.pallas import tpu as pltpu

N_DEV = 4
N_PER = 2048
D = 1024
E = 32
E_LOC = 8
CAP = 204
CAPP = 208
SLOTS = E_LOC * CAPP
R_ROWS = N_PER // 128


def _ring_ag(x, r2):

    def body(x_ref, r_ref, xall_ref, rall_ref, sx, rx, sr, rr):
        my = lax.axis_index("i")
        left = lax.rem(my + N_DEV - 1, N_DEV)
        right = lax.rem(my + 1, N_DEV)

        barrier_sem = pltpu.get_barrier_semaphore()
        for nbr in (left, right):
            pl.semaphore_signal(
                barrier_sem, inc=1, device_id=(nbr,),
                device_id_type=pl.DeviceIdType.MESH,
            )
        pl.semaphore_wait(barrier_sem, 2)

        xall_ref[pl.ds(my * N_PER, N_PER), :] = x_ref[:, :].astype(jnp.bfloat16)
        rall_ref[pl.ds(my * R_ROWS, R_ROWS), :] = r_ref[:, :]

        HD = D // 2

        def xat(b, c0, cw):
            return xall_ref.at[pl.ds(b * N_PER, N_PER), pl.ds(c0, cw)]

        def rat(b):
            return rall_ref.at[pl.ds(b * R_ROWS, R_ROWS), :]

        def rdma(src, dst, ssem, rsem, dev):
            return pltpu.make_async_remote_copy(
                src_ref=src, dst_ref=dst, send_sem=ssem, recv_sem=rsem,
                device_id=(dev,), device_id_type=pl.DeviceIdType.MESH,
            )

        hop1 = [
            rdma(xat(my, 0, D), xat(my, 0, D), sx.at[0], rx.at[0], right),
            rdma(xat(my, 0, D), xat(my, 0, D), sx.at[1], rx.at[1], left),
            rdma(rat(my), rat(my), sr.at[0], rr.at[0], right),
            rdma(rat(my), rat(my), sr.at[1], rr.at[1], left),
        ]
        for op in hop1:
            op.start()
        for op in hop1:
            op.wait()

        hop2 = [
            rdma(xat(left, 0, HD), xat(left, 0, HD), sx.at[2], rx.at[2], right),
            rdma(xat(right, HD, HD), xat(right, HD, HD), sx.at[3], rx.at[3], left),
            rdma(rat(left), rat(left), sr.at[2], rr.at[2], right),
        ]
        for op in hop2:
            op.start()
        for op in hop2:
            op.wait()

    return pl.pallas_call(
        body,
        out_shape=(
            jax.ShapeDtypeStruct((N_DEV * N_PER, D), jnp.bfloat16),
            jax.ShapeDtypeStruct((N_DEV * R_ROWS, 128), jnp.int32),
        ),
        in_specs=[
            pl.BlockSpec(memory_space=pltpu.VMEM),
            pl.BlockSpec(memory_space=pltpu.VMEM),
        ],
        out_specs=(
            pl.BlockSpec(memory_space=pltpu.VMEM),
            pl.BlockSpec(memory_space=pltpu.VMEM),
        ),
        scratch_shapes=[
            pltpu.SemaphoreType.DMA((4,)),
            pltpu.SemaphoreType.DMA((4,)),
            pltpu.SemaphoreType.DMA((3,)),
            pltpu.SemaphoreType.DMA((3,)),
        ],
        compiler_params=pltpu.CompilerParams(collective_id=0),
    )(x, r2)


def _expert_mm_ag(x_all, rr2, rk2, expert_W):

    def body(x_ref, rr_ref, rk_ref, ew_ref, yall_ref, wbuf, csem, ss, rs):
        my = lax.axis_index("i")
        left = lax.rem(my + N_DEV - 1, N_DEV)
        right = lax.rem(my + 1, N_DEV)

        barrier_sem = pltpu.get_barrier_semaphore()
        for nbr in (left, right):
            pl.semaphore_signal(
                barrier_sem, inc=1, device_id=(nbr,),
                device_id_type=pl.DeviceIdType.MESH,
            )
        pl.semaphore_wait(barrier_sem, 2)

        def expert_step(le, _):
            cp = pltpu.make_async_copy(ew_ref.at[le], wbuf, csem)
            cp.start()
            e = my * E_LOC + le
            rr = rr_ref[0:1, :]
            rk = rk_ref[0:1, :]
            c_iota = lax.broadcasted_iota(jnp.int32, (CAPP, N_DEV * N_PER), 0)
            mine = (rr == e) & (rk < CAP)
            onehot = ((c_iota == rk) & mine).astype(jnp.bfloat16)
            xd = jnp.dot(
                onehot, x_ref[:, :], preferred_element_type=jnp.float32
            ).astype(jnp.bfloat16)
            cp.wait()
            w16 = wbuf[:, :].astype(jnp.bfloat16)
            y = jnp.dot(xd, w16, preferred_element_type=jnp.float32)
            yall_ref[pl.ds(my * SLOTS + le * CAPP, CAPP), :] = y.astype(jnp.bfloat16)
            return 0

        lax.fori_loop(0, E_LOC, expert_step, 0)

        HD = D // 2

        def yat(b, c0, cw):
            return yall_ref.at[pl.ds(b * SLOTS, SLOTS), pl.ds(c0, cw)]

        def rdma(src, ssem, rsem, dev):
            return pltpu.make_async_remote_copy(
                src_ref=src, dst_ref=src, send_sem=ssem, recv_sem=rsem,
                device_id=(dev,), device_id_type=pl.DeviceIdType.MESH,
            )

        hop1 = [
            rdma(yat(my, 0, D), ss.at[0], rs.at[0], right),
            rdma(yat(my, 0, D), ss.at[1], rs.at[1], left),
        ]
        for op in hop1:
            op.start()
        for op in hop1:
            op.wait()
        hop2 = [
            rdma(yat(left, 0, HD), ss.at[2], rs.at[2], right),
            rdma(yat(right, HD, HD), ss.at[3], rs.at[3], left),
        ]
        for op in hop2:
            op.start()
        for op in hop2:
            op.wait()

    return pl.pallas_call(
        body,
        out_shape=jax.ShapeDtypeStruct((N_DEV * SLOTS, D), jnp.bfloat16),
        in_specs=[
            pl.BlockSpec(memory_space=pltpu.VMEM),
            pl.BlockSpec(memory_space=pltpu.VMEM),
            pl.BlockSpec(memory_space=pltpu.VMEM),
            pl.BlockSpec(memory_space=pl.ANY),
        ],
        out_specs=pl.BlockSpec(memory_space=pltpu.VMEM),
        scratch_shapes=[
            pltpu.VMEM((D, D), jnp.float32),
            pltpu.SemaphoreType.DMA,
            pltpu.SemaphoreType.DMA((4,)),
            pltpu.SemaphoreType.DMA((4,)),
        ],
        compiler_params=pltpu.CompilerParams(collective_id=1),
    )(x_all, rr2, rk2, expert_W)


def _combine(y_all, gm2):

    def body(y_ref, gm_ref, out_ref):
        gmv = gm_ref[:, 0:1]
        out_ref[:, :] = jnp.zeros((N_PER, D), jnp.float32)

        def step(s, _):
            k = lax.broadcasted_iota(jnp.int32, (N_PER, SLOTS), 1) + s * SLOTS
            G = (gmv == k).astype(jnp.bfloat16)
            out_ref[:, :] += jnp.dot(
                G, y_ref[pl.ds(s * SLOTS, SLOTS), :],
                preferred_element_type=jnp.float32,
            )
            return 0

        lax.fori_loop(0, N_DEV, step, 0)

    return pl.pallas_call(
        body,
        out_shape=jax.ShapeDtypeStruct((N_PER, D), jnp.float32),
        in_specs=[
            pl.BlockSpec(memory_space=pltpu.VMEM),
            pl.BlockSpec(memory_space=pltpu.VMEM),
        ],
        out_specs=pl.BlockSpec(memory_space=pltpu.VMEM),
    )(y_all, gm2)


def kernel(x, router_W, route_idx, expert_W):
    del router_W
    my = lax.axis_index("i")

    x_all, rall = _ring_ag(x, route_idx.reshape(R_ROWS, 128))
    r_all = rall.reshape(N_DEV * N_PER)

    oh = (r_all[:, None] == jnp.arange(E, dtype=jnp.int32)[None, :]).astype(jnp.int32)
    rank = jnp.sum((jnp.cumsum(oh, axis=0) - 1) * oh, axis=1)

    rr2 = r_all.reshape(1, N_DEV * N_PER)
    rk2 = rank.reshape(1, N_DEV * N_PER)
    y_all = _expert_mm_ag(x_all, rr2, rk2, expert_W)

    r_mine = lax.dynamic_slice(r_all, (my * N_PER,), (N_PER,))
    rank_mine = lax.dynamic_slice(rank, (my * N_PER,), (N_PER,))
    kept_mine = rank_mine < CAP
    gm = jnp.where(kept_mine, r_mine * CAPP + rank_mine, N_DEV * SLOTS)
    return _combine(y_all, gm.reshape(N_PER, 1))


# device time: 214597 ns/iter; 1.7594x vs baseline; 1.1690x over previous
import jax
import jax.numpy as jnp
from jax import lax
from jax.experimental import pallas as pl
from jax.experimental.pallas import tpu as pltpu

N_DEV = 4
N_PER = 2048
D = 1024
E = 32
E_LOC = 8
CAP = 204
CAPP = 208
SLOTS = E_LOC * CAPP
R_ROWS = N_PER // 128


def _ring_ag(x, r2):

    def body(x_ref, r_ref, xall_ref, rall_ref, sx, rx, sr, rr):
        my = lax.axis_index("i")
        left = lax.rem(my + N_DEV - 1, N_DEV)
        right = lax.rem(my + 1, N_DEV)

        barrier_sem = pltpu.get_barrier_semaphore()
        for nbr in (left, right):
            pl.semaphore_signal(
                barrier_sem, inc=1, device_id=(nbr,),
                device_id_type=pl.DeviceIdType.MESH,
            )
        pl.semaphore_wait(barrier_sem, 2)

        xall_ref[pl.ds(my * N_PER, N_PER), :] = x_ref[:, :].astype(jnp.bfloat16)
        rall_ref[pl.ds(my * R_ROWS, R_ROWS), :] = r_ref[:, :]

        HD = D // 2

        def xat(b, c0, cw):
            return xall_ref.at[pl.ds(b * N_PER, N_PER), pl.ds(c0, cw)]

        def rat(b):
            return rall_ref.at[pl.ds(b * R_ROWS, R_ROWS), :]

        def rdma(src, dst, ssem, rsem, dev):
            return pltpu.make_async_remote_copy(
                src_ref=src, dst_ref=dst, send_sem=ssem, recv_sem=rsem,
                device_id=(dev,), device_id_type=pl.DeviceIdType.MESH,
            )

        hop1 = [
            rdma(xat(my, 0, D), xat(my, 0, D), sx.at[0], rx.at[0], right),
            rdma(xat(my, 0, D), xat(my, 0, D), sx.at[1], rx.at[1], left),
            rdma(rat(my), rat(my), sr.at[0], rr.at[0], right),
            rdma(rat(my), rat(my), sr.at[1], rr.at[1], left),
        ]
        for op in hop1:
            op.start()
        for op in hop1:
            op.wait()

        hop2 = [
            rdma(xat(left, 0, HD), xat(left, 0, HD), sx.at[2], rx.at[2], right),
            rdma(xat(right, HD, HD), xat(right, HD, HD), sx.at[3], rx.at[3], left),
            rdma(rat(left), rat(left), sr.at[2], rr.at[2], right),
        ]
        for op in hop2:
            op.start()
        for op in hop2:
            op.wait()

    return pl.pallas_call(
        body,
        out_shape=(
            jax.ShapeDtypeStruct((N_DEV * N_PER, D), jnp.bfloat16),
            jax.ShapeDtypeStruct((N_DEV * R_ROWS, 128), jnp.int32),
        ),
        in_specs=[
            pl.BlockSpec(memory_space=pltpu.VMEM),
            pl.BlockSpec(memory_space=pltpu.VMEM),
        ],
        out_specs=(
            pl.BlockSpec(memory_space=pltpu.VMEM),
            pl.BlockSpec(memory_space=pltpu.VMEM),
        ),
        scratch_shapes=[
            pltpu.SemaphoreType.DMA((4,)),
            pltpu.SemaphoreType.DMA((4,)),
            pltpu.SemaphoreType.DMA((3,)),
            pltpu.SemaphoreType.DMA((3,)),
        ],
        compiler_params=pltpu.CompilerParams(collective_id=0),
    )(x, r2)


def _expert_mm_ag_combine(x_all, rr2, rk2, gm2, expert_W):
    HALF = SLOTS // 2
    HD = D // 2

    def body(x_ref, rr_ref, rk_ref, gm_ref, ew_ref, out_ref, yall_ref,
             wbuf, csem, ss, rs):
        my = lax.axis_index("i")
        left = lax.rem(my + N_DEV - 1, N_DEV)
        right = lax.rem(my + 1, N_DEV)

        barrier_sem = pltpu.get_barrier_semaphore()
        for nbr in (left, right):
            pl.semaphore_signal(
                barrier_sem, inc=1, device_id=(nbr,),
                device_id_type=pl.DeviceIdType.MESH,
            )
        pl.semaphore_wait(barrier_sem, 2)

        def expert_step(le, _):
            cp = pltpu.make_async_copy(ew_ref.at[le], wbuf, csem)
            cp.start()
            e = my * E_LOC + le
            rr = rr_ref[0:1, :]
            rk = rk_ref[0:1, :]
            rk16 = rk.astype(jnp.int16)
            c_iota = lax.broadcasted_iota(jnp.int16, (CAPP, N_DEV * N_PER), 0)
            mine = (rr == e) & (rk < CAP)
            onehot = ((c_iota == rk16) & mine).astype(jnp.bfloat16)
            xd = jnp.dot(
                onehot, x_ref[:, :], preferred_element_type=jnp.float32
            ).astype(jnp.bfloat16)
            cp.wait()
            w16 = wbuf[:, :].astype(jnp.bfloat16)
            y = jnp.dot(xd, w16, preferred_element_type=jnp.float32)
            yall_ref[pl.ds(my * SLOTS + le * CAPP, CAPP), :] = y.astype(jnp.bfloat16)
            return 0

        def rdma(src, ssem, rsem, dev):
            return pltpu.make_async_remote_copy(
                src_ref=src, dst_ref=src, send_sem=ssem, recv_sem=rsem,
                device_id=(dev,), device_id_type=pl.DeviceIdType.MESH,
            )

        def yrows(r0, nr, c0=0, cw=D):
            return yall_ref.at[pl.ds(r0, nr), pl.ds(c0, cw)]

        lax.fori_loop(0, E_LOC // 2, expert_step, 0)
        hop1 = [
            rdma(yrows(my * SLOTS, HALF), ss.at[0], rs.at[0], right),
            rdma(yrows(my * SLOTS, HALF), ss.at[1], rs.at[1], left),
        ]
        for op in hop1[:2]:
            op.start()
        lax.fori_loop(E_LOC // 2, E_LOC, expert_step, 0)
        hop1 += [
            rdma(yrows(my * SLOTS + HALF, HALF), ss.at[2], rs.at[2], right),
            rdma(yrows(my * SLOTS + HALF, HALF), ss.at[3], rs.at[3], left),
        ]
        for op in hop1[2:]:
            op.start()

        gmv = gm_ref[:, 0:1].astype(jnp.int16)
        k_iota = lax.broadcasted_iota(jnp.int16, (N_PER, SLOTS), 1)

        def combine_chunk(s):
            base = (s * SLOTS).astype(jnp.int16)
            G = (gmv == k_iota + base).astype(jnp.bfloat16)
            out_ref[:, :] += jnp.dot(
                G, yall_ref[pl.ds(s * SLOTS, SLOTS), :],
                preferred_element_type=jnp.float32,
            ).astype(jnp.bfloat16)

        out_ref[:, :] = jnp.zeros((N_PER, D), jnp.bfloat16)
        combine_chunk(my)

        for op in hop1:
            op.wait()

        hop2 = [
            rdma(yrows(left * SLOTS, SLOTS, 0, HD), ss.at[4], rs.at[4], right),
            rdma(yrows(right * SLOTS, SLOTS, HD, HD), ss.at[5], rs.at[5], left),
        ]
        for op in hop2:
            op.start()
        combine_chunk(left)
        combine_chunk(right)
        for op in hop2:
            op.wait()
        combine_chunk(lax.rem(my + 2, N_DEV))

    return pl.pallas_call(
        body,
        out_shape=jax.ShapeDtypeStruct((N_PER, D), jnp.bfloat16),
        in_specs=[
            pl.BlockSpec(memory_space=pltpu.VMEM),
            pl.BlockSpec(memory_space=pltpu.VMEM),
            pl.BlockSpec(memory_space=pltpu.VMEM),
            pl.BlockSpec(memory_space=pltpu.VMEM),
            pl.BlockSpec(memory_space=pl.ANY),
        ],
        out_specs=pl.BlockSpec(memory_space=pltpu.VMEM),
        scratch_shapes=[
            pltpu.VMEM((N_DEV * SLOTS, D), jnp.bfloat16),
            pltpu.VMEM((D, D), jnp.float32),
            pltpu.SemaphoreType.DMA,
            pltpu.SemaphoreType.DMA((6,)),
            pltpu.SemaphoreType.DMA((6,)),
        ],
        compiler_params=pltpu.CompilerParams(
            collective_id=1, vmem_limit_bytes=56 * 1024 * 1024,
        ),
    )(x_all, rr2, rk2, gm2, expert_W)


def kernel(x, router_W, route_idx, expert_W):
    del router_W
    my = lax.axis_index("i")

    x_all, rall = _ring_ag(x, route_idx.reshape(R_ROWS, 128))
    r_all = rall.reshape(N_DEV * N_PER)

    oh = (r_all[:, None] == jnp.arange(E, dtype=jnp.int32)[None, :]).astype(jnp.int32)
    rank = jnp.sum((jnp.cumsum(oh, axis=0) - 1) * oh, axis=1)

    r_mine = lax.dynamic_slice(r_all, (my * N_PER,), (N_PER,))
    rank_mine = lax.dynamic_slice(rank, (my * N_PER,), (N_PER,))
    kept_mine = rank_mine < CAP
    gm = jnp.where(kept_mine, r_mine * CAPP + rank_mine, N_DEV * SLOTS)

    rr2 = r_all.reshape(1, N_DEV * N_PER)
    rk2 = rank.reshape(1, N_DEV * N_PER)
    return _expert_mm_ag_combine(
        x_all, rr2, rk2, gm.reshape(N_PER, 1), expert_W
    ).astype(jnp.float32)


# device time: 194267 ns/iter; 1.9435x vs baseline; 1.1046x over previous
import jax
import jax.numpy as jnp
from jax import lax
from jax.experimental import pallas as pl
from jax.experimental.pallas import tpu as pltpu

N_DEV = 4
N_PER = 2048
D = 1024
E = 32
E_LOC = 8
CAP = 204
CAPP = 208
SLOTS = E_LOC * CAPP
R_ROWS = N_PER // 128
HD = D // 2
HALF = SLOTS // 2


def _route_ag(r2):

    def body(r_ref, rall_ref, sr, rr):
        my = lax.axis_index("i")
        left = lax.rem(my + N_DEV - 1, N_DEV)
        right = lax.rem(my + 1, N_DEV)

        barrier_sem = pltpu.get_barrier_semaphore()
        for nbr in (left, right):
            pl.semaphore_signal(
                barrier_sem, inc=1, device_id=(nbr,),
                device_id_type=pl.DeviceIdType.MESH,
            )
        pl.semaphore_wait(barrier_sem, 2)

        rall_ref[pl.ds(my * R_ROWS, R_ROWS), :] = r_ref[:, :]
        for h in range(N_DEV - 1):
            sb = lax.rem(my + N_DEV - h, N_DEV)
            op = pltpu.make_async_remote_copy(
                src_ref=rall_ref.at[pl.ds(sb * R_ROWS, R_ROWS), :],
                dst_ref=rall_ref.at[pl.ds(sb * R_ROWS, R_ROWS), :],
                send_sem=sr.at[h], recv_sem=rr.at[h],
                device_id=(right,), device_id_type=pl.DeviceIdType.MESH,
            )
            op.start()
            op.wait()

    return pl.pallas_call(
        body,
        out_shape=jax.ShapeDtypeStruct((N_DEV * R_ROWS, 128), jnp.int32),
        in_specs=[pl.BlockSpec(memory_space=pltpu.VMEM)],
        out_specs=pl.BlockSpec(memory_space=pltpu.VMEM),
        scratch_shapes=[
            pltpu.SemaphoreType.DMA((N_DEV - 1,)),
            pltpu.SemaphoreType.DMA((N_DEV - 1,)),
        ],
        compiler_params=pltpu.CompilerParams(collective_id=0),
    )(r2)


def _moe_fused(x16, rr2, rk2, gm2, expert_W):

    def body(x_ref, rr_ref, rk_ref, gm_ref, ew_ref, out_ref,
             xall_ref, yall_ref, wbuf, csem, sx, rx, ss, rs):
        my = lax.axis_index("i")
        left = lax.rem(my + N_DEV - 1, N_DEV)
        right = lax.rem(my + 1, N_DEV)
        opp = lax.rem(my + 2, N_DEV)

        barrier_sem = pltpu.get_barrier_semaphore()
        for nbr in (left, right):
            pl.semaphore_signal(
                barrier_sem, inc=1, device_id=(nbr,),
                device_id_type=pl.DeviceIdType.MESH,
            )
        pl.semaphore_wait(barrier_sem, 2)

        def rdma(src, ssem, rsem, dev):
            return pltpu.make_async_remote_copy(
                src_ref=src, dst_ref=src, send_sem=ssem, recv_sem=rsem,
                device_id=(dev,), device_id_type=pl.DeviceIdType.MESH,
            )

        def xat(b, c0, cw):
            return xall_ref.at[pl.ds(b * N_PER, N_PER), pl.ds(c0, cw)]

        def yrows(r0, nr, c0, cw):
            return yall_ref.at[pl.ds(r0, nr), pl.ds(c0, cw)]

        xall_ref[pl.ds(my * N_PER, N_PER), :] = x_ref[:, :]
        xhop1 = [
            rdma(xat(my, 0, D), sx.at[0], rx.at[0], right),
            rdma(xat(my, 0, D), sx.at[1], rx.at[1], left),
        ]
        for op in xhop1:
            op.start()

        yall_ref[pl.ds(my * SLOTS, SLOTS), :] = jnp.zeros((SLOTS, D), jnp.bfloat16)

        def dispatch_block(b):
            rrb = rr_ref[0:1, pl.ds(b * N_PER, N_PER)]
            rkb = rk_ref[0:1, pl.ds(b * N_PER, N_PER)]
            rk16 = rkb.astype(jnp.int16)
            c_iota = lax.broadcasted_iota(jnp.int16, (CAPP, N_PER), 0)

            def le_step(le, _):
                e = my * E_LOC + le
                mine = (rrb == e) & (rkb < CAP)
                onehot = ((c_iota == rk16) & mine).astype(jnp.bfloat16)
                yall_ref[pl.ds(my * SLOTS + le * CAPP, CAPP), :] += jnp.dot(
                    onehot, xall_ref[pl.ds(b * N_PER, N_PER), :],
                    preferred_element_type=jnp.float32,
                ).astype(jnp.bfloat16)
                return 0

            lax.fori_loop(0, E_LOC, le_step, 0)

        dispatch_block(my)
        for op in xhop1:
            op.wait()
        xhop2 = [
            rdma(xat(left, 0, HD), sx.at[2], rx.at[2], right),
            rdma(xat(right, HD, HD), sx.at[3], rx.at[3], left),
        ]
        for op in xhop2:
            op.start()
        dispatch_block(left)
        dispatch_block(right)
        for op in xhop2:
            op.wait()
        dispatch_block(opp)

        def mm_step(le, _):
            cp = pltpu.make_async_copy(ew_ref.at[le], wbuf, csem)
            cp.start()
            cp.wait()
            w16 = wbuf[:, :].astype(jnp.bfloat16)
            y = jnp.dot(yall_ref[pl.ds(my * SLOTS + le * CAPP, CAPP), :], w16,
                        preferred_element_type=jnp.float32)
            yall_ref[pl.ds(my * SLOTS + le * CAPP, CAPP), :] = y.astype(jnp.bfloat16)
            return 0

        lax.fori_loop(0, E_LOC // 2, mm_step, 0)
        yhop1 = [
            rdma(yrows(my * SLOTS, HALF, 0, D), ss.at[0], rs.at[0], right),
            rdma(yrows(my * SLOTS, HALF, 0, D), ss.at[1], rs.at[1], left),
        ]
        for op in yhop1[:2]:
            op.start()
        lax.fori_loop(E_LOC // 2, E_LOC, mm_step, 0)
        yhop1 += [
            rdma(yrows(my * SLOTS + HALF, HALF, 0, D), ss.at[2], rs.at[2], right),
            rdma(yrows(my * SLOTS + HALF, HALF, 0, D), ss.at[3], rs.at[3], left),
        ]
        for op in yhop1[2:]:
            op.start()

        gmv = gm_ref[:, 0:1].astype(jnp.int16)
        k_iota = lax.broadcasted_iota(jnp.int16, (N_PER, SLOTS), 1)

        def combine_chunk(s):
            base = (s * SLOTS).astype(jnp.int16)
            G = (gmv == k_iota + base).astype(jnp.bfloat16)
            out_ref[:, :] += jnp.dot(
                G, yall_ref[pl.ds(s * SLOTS, SLOTS), :],
                preferred_element_type=jnp.float32,
            ).astype(jnp.bfloat16)

        out_ref[:, :] = jnp.zeros((N_PER, D), jnp.bfloat16)
        combine_chunk(my)
        for op in yhop1:
            op.wait()
        yhop2 = [
            rdma(yrows(left * SLOTS, SLOTS, 0, HD), ss.at[4], rs.at[4], right),
            rdma(yrows(right * SLOTS, SLOTS, HD, HD), ss.at[5], rs.at[5], left),
        ]
        for op in yhop2:
            op.start()
        combine_chunk(left)
        combine_chunk(right)
        for op in yhop2:
            op.wait()
        combine_chunk(opp)

    return pl.pallas_call(
        body,
        out_shape=jax.ShapeDtypeStruct((N_PER, D), jnp.bfloat16),
        in_specs=[
            pl.BlockSpec(memory_space=pltpu.VMEM),
            pl.BlockSpec(memory_space=pltpu.VMEM),
            pl.BlockSpec(memory_space=pltpu.VMEM),
            pl.BlockSpec(memory_space=pltpu.VMEM),
            pl.BlockSpec(memory_space=pl.ANY),
        ],
        out_specs=pl.BlockSpec(memory_space=pltpu.VMEM),
        scratch_shapes=[
            pltpu.VMEM((N_DEV * N_PER, D), jnp.bfloat16),
            pltpu.VMEM((N_DEV * SLOTS, D), jnp.bfloat16),
            pltpu.VMEM((D, D), jnp.float32),
            pltpu.SemaphoreType.DMA,
            pltpu.SemaphoreType.DMA((4,)),
            pltpu.SemaphoreType.DMA((4,)),
            pltpu.SemaphoreType.DMA((6,)),
            pltpu.SemaphoreType.DMA((6,)),
        ],
        compiler_params=pltpu.CompilerParams(
            collective_id=1, vmem_limit_bytes=58 * 1024 * 1024,
        ),
    )(x16, rr2, rk2, gm2, expert_W)


def kernel(x, router_W, route_idx, expert_W):
    del router_W
    my = lax.axis_index("i")

    rall = _route_ag(route_idx.reshape(R_ROWS, 128))
    r_all = rall.reshape(N_DEV * N_PER)
    oh = (r_all[:, None] == jnp.arange(E, dtype=jnp.int32)[None, :]).astype(jnp.int32)
    rank = jnp.sum((jnp.cumsum(oh, axis=0) - 1) * oh, axis=1)

    r_mine = lax.dynamic_slice(r_all, (my * N_PER,), (N_PER,))
    rank_mine = lax.dynamic_slice(rank, (my * N_PER,), (N_PER,))
    kept_mine = rank_mine < CAP
    gm = jnp.where(kept_mine, r_mine * CAPP + rank_mine, N_DEV * SLOTS)

    out = _moe_fused(
        x.astype(jnp.bfloat16),
        r_all.reshape(1, N_DEV * N_PER),
        rank.reshape(1, N_DEV * N_PER),
        gm.reshape(N_PER, 1),
        expert_W,
    )
    return out.astype(jnp.float32)
